# baseline (speedup 1.0000x reference)
import dataclasses

import jax
import jax.numpy as jnp
from jax import lax
from jax.experimental import pallas as pl
from jax.experimental.pallas import tpu as pltpu
from jax.experimental.pallas import tpu_sc as plsc

CAP = 262144
D = 64
B = 64
TOPK = 4
THR = 0.1
EPS = 1e-8

KB = 8192
NSTEP = CAP // KB
CHUNK = 128
NCHUNK = CAP // CHUNK
CPS = KB // CHUNK
NCAND = TOPK * CHUNK

NEG_INF = float("-inf")
BIG_I32 = 1 << 30


def _normalize_rows(x):
    nrm = jnp.sqrt(jnp.sum(x * x, axis=1))
    return x / jnp.maximum(nrm, EPS)[:, None]


def _sims_bf16(kn, qn):
    return lax.dot_general(
        kn.astype(jnp.bfloat16), qn.astype(jnp.bfloat16),
        (((1,), (1,)), ((), ())), preferred_element_type=jnp.float32)


def _p1_body(q_ref, k_ref, cids_ref, acc_ref):
    i = pl.program_id(0)
    k = k_ref[...]
    kn = _normalize_rows(k)
    qn = _normalize_rows(q_ref[...])
    s = _sims_bf16(kn, qn)
    cm = jnp.max(s.reshape(CPS, CHUNK, B), axis=1)
    acc_ref[pl.ds(i * CPS, CPS), :] = cm

    @pl.when(i == NSTEP - 1)
    def _():
        a = acc_ref[...]
        rows = lax.broadcasted_iota(jnp.int32, (NCHUNK, B), 0)
        picks = []
        for _t in range(TOPK):
            m = jnp.max(a, axis=0)
            pick = jnp.min(jnp.where(a == m[None, :], rows, BIG_I32), axis=0)
            a = jnp.where(rows == pick[None, :], NEG_INF, a)
            picks.append(pick)
        picks += [jnp.zeros((B,), jnp.int32)] * 4
        cids_ref[...] = jnp.stack(picks, axis=0)


def _phase1(query, memory_keys):
    return pl.pallas_call(
        _p1_body,
        grid=(NSTEP,),
        in_specs=[
            pl.BlockSpec((B, D), lambda i: (0, 0)),
            pl.BlockSpec((KB, D), lambda i: (i, 0)),
        ],
        out_specs=pl.BlockSpec((8, B), lambda i: (0, 0)),
        out_shape=jax.ShapeDtypeStruct((8, B), jnp.int32),
        scratch_shapes=[pltpu.VMEM((NCHUNK, B), jnp.float32)],
    )(query, memory_keys)


RPS = 8
NP2 = B // RPS
CCH = RPS * TOPK


def _p2_body(ids_ref, q_ref, kh_ref, vh_ref,
             tidx_ref, rsim_ref, valf_ref, rkeys_ref, rvals_ref,
             s_ref, kbuf, vbuf, sem):
    i = pl.program_id(0)
    for j in range(CCH):
        base = ids_ref[CCH * i + j] * CHUNK
        pltpu.make_async_copy(
            kh_ref.at[pl.ds(base, CHUNK), :],
            kbuf.at[pl.ds(j * CHUNK, CHUNK), :], sem).start()
        pltpu.make_async_copy(
            vh_ref.at[pl.ds(base, CHUNK), :],
            vbuf.at[pl.ds(j * CHUNK, CHUNK), :], sem).start()
    for j in range(CCH):
        base = ids_ref[CCH * i + j] * CHUNK
        pltpu.make_async_copy(
            kh_ref.at[pl.ds(base, CHUNK), :],
            kbuf.at[pl.ds(j * CHUNK, CHUNK), :], sem).wait()
        pltpu.make_async_copy(
            vh_ref.at[pl.ds(base, CHUNK), :],
            vbuf.at[pl.ds(j * CHUNK, CHUNK), :], sem).wait()
    kc = kbuf[...]
    kn = _normalize_rows(kc)
    qn = _normalize_rows(q_ref[...])
    s_ref[...] = lax.dot_general(
        qn.astype(jnp.bfloat16), kn.astype(jnp.bfloat16),
        (((1,), (1,)), ((), ())), preferred_element_type=jnp.float32)

    lane = lax.broadcasted_iota(jnp.int32, (1, NCAND), 1)
    slot = lane >> 7
    offs = lane & (CHUNK - 1)
    picks_a, sims_a, vals_a, krows_a, vrows_a = [], [], [], [], []
    for r in range(RPS):
        val = s_ref[pl.ds(i * RPS + r, 1), pl.ds(r * NCAND, NCAND)]
        base = jnp.where(
            slot == 0, ids_ref[CCH * i + 4 * r],
            jnp.where(slot == 1, ids_ref[CCH * i + 4 * r + 1],
                      jnp.where(slot == 2, ids_ref[CCH * i + 4 * r + 2],
                                ids_ref[CCH * i + 4 * r + 3])))
        flat = base * CHUNK + offs
        kcr = kc[r * NCAND:(r + 1) * NCAND]
        vcr = vbuf[pl.ds(r * NCAND, NCAND), :]

        mval = jnp.where(val >= THR, val, NEG_INF)
        picks, sims_o, vals_o, krows, vrows = [], [], [], [], []
        for _t in range(TOPK):
            m = jnp.max(mval)
            pick = jnp.min(jnp.where(mval == m, flat, BIG_I32))
            pick = jnp.where(m == NEG_INF, jnp.int32(0), pick)
            ok = m >= THR
            hit = jnp.logical_and(mval == m, flat == pick)
            mval = jnp.where(flat == pick, NEG_INF, mval)
            picks.append(pick)
            sims_o.append(jnp.where(ok, m, 0.0))
            vals_o.append(jnp.where(ok, 1.0, 0.0))
            selt = jnp.transpose(jnp.logical_and(hit, ok))
            krows.append(jnp.sum(jnp.where(selt, kcr, 0.0),
                                 axis=0, keepdims=True))
            vrows.append(jnp.sum(jnp.where(selt, vcr, 0.0),
                                 axis=0, keepdims=True))
        picks_a.append(jnp.stack(picks).reshape(1, 1, TOPK))
        sims_a.append(jnp.stack(sims_o).reshape(1, 1, TOPK))
        vals_a.append(jnp.stack(vals_o).reshape(1, 1, TOPK))
        krows_a.append(jnp.concatenate(krows, axis=0).reshape(1, TOPK, D))
        vrows_a.append(jnp.concatenate(vrows, axis=0).reshape(1, TOPK, D))
    tidx_ref[...] = jnp.concatenate(picks_a, axis=0)
    rsim_ref[...] = jnp.concatenate(sims_a, axis=0).astype(jnp.float32)
    valf_ref[...] = jnp.concatenate(vals_a, axis=0).astype(jnp.float32)
    rkeys_ref[...] = jnp.concatenate(krows_a, axis=0)
    rvals_ref[...] = jnp.concatenate(vrows_a, axis=0)


def _phase2(query, memory_keys, memory_values, cids_flat):
    grid_spec = pltpu.PrefetchScalarGridSpec(
        num_scalar_prefetch=1,
        grid=(NP2,),
        in_specs=[
            pl.BlockSpec((B, D), lambda i, ids: (0, 0)),
            pl.BlockSpec(memory_space=pl.ANY),
            pl.BlockSpec(memory_space=pl.ANY),
        ],
        out_specs=[
            pl.BlockSpec((RPS, 1, TOPK), lambda i, ids: (i, 0, 0)),
            pl.BlockSpec((RPS, 1, TOPK), lambda i, ids: (i, 0, 0)),
            pl.BlockSpec((RPS, 1, TOPK), lambda i, ids: (i, 0, 0)),
            pl.BlockSpec((RPS, TOPK, D), lambda i, ids: (i, 0, 0)),
            pl.BlockSpec((RPS, TOPK, D), lambda i, ids: (i, 0, 0)),
        ],
        scratch_shapes=[
            pltpu.VMEM((B, CCH * CHUNK), jnp.float32),
            pltpu.VMEM((CCH * CHUNK, D), jnp.float32),
            pltpu.VMEM((CCH * CHUNK, D), jnp.float32),
            pltpu.SemaphoreType.DMA,
        ],
    )
    return pl.pallas_call(
        _p2_body,
        grid_spec=grid_spec,
        out_shape=[
            jax.ShapeDtypeStruct((B, 1, TOPK), jnp.int32),
            jax.ShapeDtypeStruct((B, 1, TOPK), jnp.float32),
            jax.ShapeDtypeStruct((B, 1, TOPK), jnp.float32),
            jax.ShapeDtypeStruct((B, TOPK, D), jnp.float32),
            jax.ShapeDtypeStruct((B, TOPK, D), jnp.float32),
        ],
    )(cids_flat, query, memory_keys, memory_values)


NSLOT = B * TOPK
SPW = NSLOT // 16
UROWS = CAP // 16
CROWS = UROWS // 16


def _extractf(vec, j, iota16):
    return jnp.sum(jnp.where(iota16 == j, vec, 0.0))


def _extracti(vec, j, iota16):
    return jnp.sum(jnp.where(iota16 == j, vec, 0))


def _p3_body(usage_hbm, idx_hbm, valid_hbm, uout_hbm,
             myidx_v, rowid_v, myval_v, urows_v, oh_v, mrows_v,
             cbuf_v, mask_sh, sem):
    core = lax.axis_index("c")
    sub = lax.axis_index("s")
    iota16 = lax.iota(jnp.int32, 16)

    @pl.when(core == 0)
    def _usage():
        pltpu.sync_copy(usage_hbm.at[pl.ds(sub * CROWS, CROWS)], cbuf_v)
        pltpu.sync_copy(cbuf_v, uout_hbm.at[pl.ds(sub * CROWS, CROWS)])

        pltpu.sync_copy(idx_hbm.at[pl.ds(sub * SPW, SPW)], myidx_v)
        pltpu.sync_copy(valid_hbm.at[pl.ds(sub * SPW, SPW)], myval_v)
        idx16 = myidx_v[...]
        vvec = myval_v[...]
        lanes = jnp.bitwise_and(idx16, 15)
        rowid_v[...] = lax.shift_right_logical(idx16, 4)
        pltpu.async_copy(usage_hbm.at[rowid_v], urows_v, sem).wait()

        zero16 = jnp.zeros((16,), jnp.float32)
        for j in range(SPW):
            oh_v[j, :] = zero16
        pltpu.sync_copy(oh_v, mask_sh.at[rowid_v])
        plsc.subcore_barrier()
        for j in range(SPW):
            lane = _extracti(lanes, j, iota16)
            vj = _extractf(vvec, j, iota16)
            oh_v[j, :] = jnp.where(iota16 == lane, vj, 0.0)
        pltpu.sync_copy(oh_v, mask_sh.at[rowid_v], add=True)
        plsc.subcore_barrier()
        pltpu.async_copy(mask_sh.at[rowid_v], mrows_v, sem).wait()

        for j in range(SPW):
            u = urows_v[j, :]
            upd = jnp.minimum(u * jnp.float32(1.1), jnp.float32(2.0))
            urows_v[j, :] = jnp.where(mrows_v[j, :] > 0.5, upd, u)
        pltpu.sync_copy(urows_v, uout_hbm.at[rowid_v])


def _phase3(usage2d, idx_flat, valid_flat):
    mesh = plsc.VectorSubcoreMesh(core_axis_name="c", subcore_axis_name="s")
    cp = pltpu.CompilerParams()
    for fld, val in (("needs_layout_passes", False),
                     ("use_tc_tiling_on_sc", False)):
        if fld in pltpu.CompilerParams.__dataclass_fields__:
            cp = dataclasses.replace(cp, **{fld: val})
    kfn = pl.kernel(
        _p3_body,
        mesh=mesh,
        compiler_params=cp,
        out_type=[
            jax.ShapeDtypeStruct((UROWS, 16), jnp.float32),
        ],
        scratch_types=[
            pltpu.VMEM((SPW,), jnp.int32),
            pltpu.VMEM((SPW,), jnp.int32),
            pltpu.VMEM((SPW,), jnp.float32),
            pltpu.VMEM((SPW, 16), jnp.float32),
            pltpu.VMEM((SPW, 16), jnp.float32),
            pltpu.VMEM((SPW, 16), jnp.float32),
            pltpu.VMEM((CROWS, 16), jnp.float32),
            pltpu.VMEM_SHARED((UROWS, 16), jnp.float32),
            pltpu.SemaphoreType.DMA,
        ],
    )
    return kfn(usage2d, idx_flat, valid_flat)[0]


def kernel(query, memory_keys, memory_values, memory_usage):
    cids8 = _phase1(query, memory_keys)
    cids_flat = jnp.transpose(cids8[:TOPK]).reshape(-1)
    tidx, rsim, valf, retrieved_keys, retrieved_values = _phase2(
        query, memory_keys, memory_values, cids_flat)
    retrieved_sims = rsim.reshape(B, TOPK)
    valid_flat = valf.reshape(-1)
    idx_flat = tidx.reshape(-1)
    usage2d = memory_usage.reshape(UROWS, 16)
    uout = _phase3(usage2d, idx_flat, valid_flat)
    new_usage = uout.reshape(CAP)
    return retrieved_values, retrieved_keys, retrieved_sims, new_usage


# speedup vs baseline: 1.1129x; 1.1129x over previous
import dataclasses

import jax
import jax.numpy as jnp
from jax import lax
from jax.experimental import pallas as pl
from jax.experimental.pallas import tpu as pltpu
from jax.experimental.pallas import tpu_sc as plsc

CAP = 262144
D = 64
B = 64
TOPK = 4
THR = 0.1
EPS = 1e-8

KB = 8192
NSTEP = CAP // KB
CHUNK = 128
NCHUNK = CAP // CHUNK
CPS = KB // CHUNK
NCAND = TOPK * CHUNK

NEG_INF = float("-inf")
BIG_I32 = 1 << 30


def _normalize_rows(x):
    nrm = jnp.sqrt(jnp.sum(x * x, axis=1))
    return x / jnp.maximum(nrm, EPS)[:, None]


def _sims_bf16(kn, qn):
    return lax.dot_general(
        kn.astype(jnp.bfloat16), qn.astype(jnp.bfloat16),
        (((1,), (1,)), ((), ())), preferred_element_type=jnp.float32)


def _p1_body(q_ref, k_ref, cids_ref, acc_ref):
    i = pl.program_id(0)
    k = k_ref[...]
    kn = _normalize_rows(k)
    qn = _normalize_rows(q_ref[...])
    s = _sims_bf16(kn, qn)
    cm = jnp.max(s.reshape(CPS, CHUNK, B), axis=1)
    acc_ref[pl.ds(i * CPS, CPS), :] = cm

    @pl.when(i == NSTEP - 1)
    def _():
        a = acc_ref[...]
        rows = lax.broadcasted_iota(jnp.int32, (NCHUNK, B), 0)
        picks = []
        for _t in range(TOPK):
            m = jnp.max(a, axis=0)
            pick = jnp.min(jnp.where(a == m[None, :], rows, BIG_I32), axis=0)
            a = jnp.where(rows == pick[None, :], NEG_INF, a)
            picks.append(pick)
        picks += [jnp.zeros((B,), jnp.int32)] * 4
        cids_ref[...] = jnp.stack(picks, axis=0)


def _phase1(query, memory_keys):
    return pl.pallas_call(
        _p1_body,
        grid=(NSTEP,),
        in_specs=[
            pl.BlockSpec((B, D), lambda i: (0, 0)),
            pl.BlockSpec((KB, D), lambda i: (i, 0)),
        ],
        out_specs=pl.BlockSpec((8, B), lambda i: (0, 0)),
        out_shape=jax.ShapeDtypeStruct((8, B), jnp.int32),
        scratch_shapes=[pltpu.VMEM((NCHUNK, B), jnp.float32)],
    )(query, memory_keys)


RPS = 4
NP2 = B // RPS
CCH = RPS * TOPK


def _p2_body(ids_ref, q_ref, *refs):
    k_refs = refs[:CCH]
    tidx_ref, rsim_ref, valf_ref, rkeys_ref, s_ref = refs[CCH:]
    i = pl.program_id(0)
    kc = jnp.concatenate([r[...] for r in k_refs], axis=0)
    kn = _normalize_rows(kc)
    qn = _normalize_rows(q_ref[...])
    s_ref[...] = lax.dot_general(
        qn.astype(jnp.bfloat16), kn.astype(jnp.bfloat16),
        (((1,), (1,)), ((), ())), preferred_element_type=jnp.float32)

    lane = lax.broadcasted_iota(jnp.int32, (1, NCAND), 1)
    slot = lane >> 7
    offs = lane & (CHUNK - 1)
    picks_a, sims_a, vals_a, krows_a = [], [], [], []
    for r in range(RPS):
        val = s_ref[pl.ds(i * RPS + r, 1), pl.ds(r * NCAND, NCAND)]
        base = jnp.where(
            slot == 0, ids_ref[CCH * i + 4 * r],
            jnp.where(slot == 1, ids_ref[CCH * i + 4 * r + 1],
                      jnp.where(slot == 2, ids_ref[CCH * i + 4 * r + 2],
                                ids_ref[CCH * i + 4 * r + 3])))
        flat = base * CHUNK + offs
        kcr = kc[r * NCAND:(r + 1) * NCAND]

        mval = jnp.where(val >= THR, val, NEG_INF)
        picks, sims_o, vals_o, krows = [], [], [], []
        for _t in range(TOPK):
            m = jnp.max(mval)
            pick = jnp.min(jnp.where(mval == m, flat, BIG_I32))
            pick = jnp.where(m == NEG_INF, jnp.int32(0), pick)
            ok = m >= THR
            hit = jnp.logical_and(mval == m, flat == pick)
            mval = jnp.where(flat == pick, NEG_INF, mval)
            picks.append(pick)
            sims_o.append(jnp.where(ok, m, 0.0))
            vals_o.append(jnp.where(ok, 1.0, 0.0))
            selt = jnp.transpose(jnp.logical_and(hit, ok))
            krows.append(jnp.sum(jnp.where(selt, kcr, 0.0),
                                 axis=0, keepdims=True))
        picks_a.append(jnp.stack(picks).reshape(1, 1, TOPK))
        sims_a.append(jnp.stack(sims_o).reshape(1, 1, TOPK))
        vals_a.append(jnp.stack(vals_o).reshape(1, 1, TOPK))
        krows_a.append(jnp.concatenate(krows, axis=0).reshape(1, TOPK, D))
    tidx_ref[...] = jnp.concatenate(picks_a, axis=0)
    rsim_ref[...] = jnp.concatenate(sims_a, axis=0).astype(jnp.float32)
    valf_ref[...] = jnp.concatenate(vals_a, axis=0).astype(jnp.float32)
    rkeys_ref[...] = jnp.concatenate(krows_a, axis=0)


def _phase2(query, memory_keys, cids_flat):
    def mk_spec(j):
        return pl.BlockSpec((CHUNK, D), lambda i, ids, j=j: (ids[CCH * i + j], 0))

    grid_spec = pltpu.PrefetchScalarGridSpec(
        num_scalar_prefetch=1,
        grid=(NP2,),
        in_specs=[pl.BlockSpec((B, D), lambda i, ids: (0, 0))]
        + [mk_spec(j) for j in range(CCH)],
        out_specs=[
            pl.BlockSpec((RPS, 1, TOPK), lambda i, ids: (i, 0, 0)),
            pl.BlockSpec((RPS, 1, TOPK), lambda i, ids: (i, 0, 0)),
            pl.BlockSpec((RPS, 1, TOPK), lambda i, ids: (i, 0, 0)),
            pl.BlockSpec((RPS, TOPK, D), lambda i, ids: (i, 0, 0)),
        ],
        scratch_shapes=[pltpu.VMEM((B, CCH * CHUNK), jnp.float32)],
    )
    return pl.pallas_call(
        _p2_body,
        grid_spec=grid_spec,
        out_shape=[
            jax.ShapeDtypeStruct((B, 1, TOPK), jnp.int32),
            jax.ShapeDtypeStruct((B, 1, TOPK), jnp.float32),
            jax.ShapeDtypeStruct((B, 1, TOPK), jnp.float32),
            jax.ShapeDtypeStruct((B, TOPK, D), jnp.float32),
        ],
    )(cids_flat, query, *([memory_keys] * CCH))


def _p2v_body(ids_ref, v0_ref, v1_ref, v2_ref, v3_ref, valf_ref, out_ref):
    rows = jnp.concatenate(
        [v0_ref[0], v1_ref[0], v2_ref[0], v3_ref[0]], axis=0)
    scale = jnp.transpose(valf_ref[0])
    out_ref[...] = (rows * scale).reshape(1, TOPK, D)


def _phase2v(memory_values, idx_flat, valf):
    vals3 = memory_values.reshape(CAP, 1, D)
    grid_spec = pltpu.PrefetchScalarGridSpec(
        num_scalar_prefetch=1,
        grid=(B,),
        in_specs=[
            pl.BlockSpec((1, 1, D), lambda i, ids: (ids[4 * i], 0, 0)),
            pl.BlockSpec((1, 1, D), lambda i, ids: (ids[4 * i + 1], 0, 0)),
            pl.BlockSpec((1, 1, D), lambda i, ids: (ids[4 * i + 2], 0, 0)),
            pl.BlockSpec((1, 1, D), lambda i, ids: (ids[4 * i + 3], 0, 0)),
            pl.BlockSpec((1, 1, TOPK), lambda i, ids: (i, 0, 0)),
        ],
        out_specs=[pl.BlockSpec((1, TOPK, D), lambda i, ids: (i, 0, 0))],
    )
    return pl.pallas_call(
        _p2v_body,
        grid_spec=grid_spec,
        out_shape=[jax.ShapeDtypeStruct((B, TOPK, D), jnp.float32)],
    )(idx_flat, vals3, vals3, vals3, vals3, valf)[0]


NSLOT = B * TOPK
SPW = NSLOT // 16
UROWS = CAP // 16
CROWS = UROWS // 16


def _extractf(vec, j, iota16):
    return jnp.sum(jnp.where(iota16 == j, vec, 0.0))


def _extracti(vec, j, iota16):
    return jnp.sum(jnp.where(iota16 == j, vec, 0))


def _p3_body(usage_hbm, idx_hbm, valid_hbm, uout_hbm,
             myidx_v, rowid_v, myval_v, urows_v, oh_v, mrows_v,
             cbuf_v, mask_sh, sem):
    core = lax.axis_index("c")
    sub = lax.axis_index("s")
    iota16 = lax.iota(jnp.int32, 16)

    @pl.when(core == 0)
    def _usage():
        pltpu.sync_copy(usage_hbm.at[pl.ds(sub * CROWS, CROWS)], cbuf_v)
        pltpu.sync_copy(cbuf_v, uout_hbm.at[pl.ds(sub * CROWS, CROWS)])

        pltpu.sync_copy(idx_hbm.at[pl.ds(sub * SPW, SPW)], myidx_v)
        pltpu.sync_copy(valid_hbm.at[pl.ds(sub * SPW, SPW)], myval_v)
        idx16 = myidx_v[...]
        vvec = myval_v[...]
        lanes = jnp.bitwise_and(idx16, 15)
        rowid_v[...] = lax.shift_right_logical(idx16, 4)
        pltpu.async_copy(usage_hbm.at[rowid_v], urows_v, sem).wait()

        zero16 = jnp.zeros((16,), jnp.float32)
        for j in range(SPW):
            oh_v[j, :] = zero16
        pltpu.sync_copy(oh_v, mask_sh.at[rowid_v])
        plsc.subcore_barrier()
        for j in range(SPW):
            lane = _extracti(lanes, j, iota16)
            vj = _extractf(vvec, j, iota16)
            oh_v[j, :] = jnp.where(iota16 == lane, vj, 0.0)
        pltpu.sync_copy(oh_v, mask_sh.at[rowid_v], add=True)
        plsc.subcore_barrier()
        pltpu.async_copy(mask_sh.at[rowid_v], mrows_v, sem).wait()

        for j in range(SPW):
            u = urows_v[j, :]
            upd = jnp.minimum(u * jnp.float32(1.1), jnp.float32(2.0))
            urows_v[j, :] = jnp.where(mrows_v[j, :] > 0.5, upd, u)
        pltpu.sync_copy(urows_v, uout_hbm.at[rowid_v])


def _phase3(usage2d, idx_flat, valid_flat):
    mesh = plsc.VectorSubcoreMesh(core_axis_name="c", subcore_axis_name="s")
    cp = pltpu.CompilerParams()
    for fld, val in (("needs_layout_passes", False),
                     ("use_tc_tiling_on_sc", False)):
        if fld in pltpu.CompilerParams.__dataclass_fields__:
            cp = dataclasses.replace(cp, **{fld: val})
    kfn = pl.kernel(
        _p3_body,
        mesh=mesh,
        compiler_params=cp,
        out_type=[
            jax.ShapeDtypeStruct((UROWS, 16), jnp.float32),
        ],
        scratch_types=[
            pltpu.VMEM((SPW,), jnp.int32),
            pltpu.VMEM((SPW,), jnp.int32),
            pltpu.VMEM((SPW,), jnp.float32),
            pltpu.VMEM((SPW, 16), jnp.float32),
            pltpu.VMEM((SPW, 16), jnp.float32),
            pltpu.VMEM((SPW, 16), jnp.float32),
            pltpu.VMEM((CROWS, 16), jnp.float32),
            pltpu.VMEM_SHARED((UROWS, 16), jnp.float32),
            pltpu.SemaphoreType.DMA,
        ],
    )
    return kfn(usage2d, idx_flat, valid_flat)[0]


def kernel(query, memory_keys, memory_values, memory_usage):
    cids8 = _phase1(query, memory_keys)
    cids_flat = jnp.transpose(cids8[:TOPK]).reshape(-1)
    tidx, rsim, valf, retrieved_keys = _phase2(query, memory_keys, cids_flat)
    retrieved_sims = rsim.reshape(B, TOPK)
    valid_flat = valf.reshape(-1)
    idx_flat = tidx.reshape(-1)
    retrieved_values = _phase2v(memory_values, idx_flat, valf)
    usage2d = memory_usage.reshape(UROWS, 16)
    uout = _phase3(usage2d, idx_flat, valid_flat)
    new_usage = uout.reshape(CAP)
    return retrieved_values, retrieved_keys, retrieved_sims, new_usage


# speedup vs baseline: 1.1813x; 1.0615x over previous
import dataclasses

import jax
import jax.numpy as jnp
from jax import lax
from jax.experimental import pallas as pl
from jax.experimental.pallas import tpu as pltpu
from jax.experimental.pallas import tpu_sc as plsc

CAP = 262144
D = 64
B = 64
TOPK = 4
THR = 0.1
EPS = 1e-8

KB = 8192
NSTEP = CAP // KB
CHUNK = 128
NCHUNK = CAP // CHUNK
CPS = KB // CHUNK
NCAND = TOPK * CHUNK

NEG_INF = float("-inf")
BIG_I32 = 1 << 30


def _normalize_rows(x):
    nrm = jnp.sqrt(jnp.sum(x * x, axis=1))
    return x / jnp.maximum(nrm, EPS)[:, None]


def _sims_bf16(kn, qn):
    return lax.dot_general(
        kn.astype(jnp.bfloat16), qn.astype(jnp.bfloat16),
        (((1,), (1,)), ((), ())), preferred_element_type=jnp.float32)


def _p1_body(q_ref, k_ref, cids_ref, acc_ref):
    i = pl.program_id(0)
    k = k_ref[...]
    kn = _normalize_rows(k)
    qn = _normalize_rows(q_ref[...])
    s = _sims_bf16(kn, qn)
    cm = jnp.max(s.reshape(CPS, CHUNK, B), axis=1)
    acc_ref[pl.ds(i * CPS, CPS), :] = cm

    @pl.when(i == NSTEP - 1)
    def _():
        a = acc_ref[...]
        rows = lax.broadcasted_iota(jnp.int32, (NCHUNK, B), 0)
        picks = []
        for _t in range(TOPK):
            m = jnp.max(a, axis=0)
            pick = jnp.min(jnp.where(a == m[None, :], rows, BIG_I32), axis=0)
            a = jnp.where(rows == pick[None, :], NEG_INF, a)
            picks.append(pick)
        picks += [jnp.zeros((B,), jnp.int32)] * 4
        cids_ref[...] = jnp.stack(picks, axis=0)


def _phase1(query, memory_keys):
    return pl.pallas_call(
        _p1_body,
        grid=(NSTEP,),
        in_specs=[
            pl.BlockSpec((B, D), lambda i: (0, 0)),
            pl.BlockSpec((KB, D), lambda i: (i, 0)),
        ],
        out_specs=pl.BlockSpec((8, B), lambda i: (0, 0)),
        out_shape=jax.ShapeDtypeStruct((8, B), jnp.int32),
        scratch_shapes=[pltpu.VMEM((NCHUNK, B), jnp.float32)],
    )(query, memory_keys)


RPS = 4
NP2 = B // RPS
CCH = RPS * TOPK


def _p2_body(ids_ref, q_ref, *refs):
    k_refs = refs[:CCH]
    tidx_ref, rsim_ref, valf_ref, rkeys_ref, s_ref = refs[CCH:]
    i = pl.program_id(0)
    kc = jnp.concatenate([r[...] for r in k_refs], axis=0)
    kn = _normalize_rows(kc)
    qn = _normalize_rows(q_ref[...])
    s_ref[...] = lax.dot_general(
        qn.astype(jnp.bfloat16), kn.astype(jnp.bfloat16),
        (((1,), (1,)), ((), ())), preferred_element_type=jnp.float32)

    lane = lax.broadcasted_iota(jnp.int32, (1, NCAND), 1)
    slot = lane >> 7
    offs = lane & (CHUNK - 1)
    picks_a, sims_a, vals_a, krows_a = [], [], [], []
    for r in range(RPS):
        val = s_ref[pl.ds(i * RPS + r, 1), pl.ds(r * NCAND, NCAND)]
        base = jnp.where(
            slot == 0, ids_ref[CCH * i + 4 * r],
            jnp.where(slot == 1, ids_ref[CCH * i + 4 * r + 1],
                      jnp.where(slot == 2, ids_ref[CCH * i + 4 * r + 2],
                                ids_ref[CCH * i + 4 * r + 3])))
        flat = base * CHUNK + offs
        kcr = kc[r * NCAND:(r + 1) * NCAND]

        mval = jnp.where(val >= THR, val, NEG_INF)
        picks, sims_o, vals_o, krows = [], [], [], []
        for _t in range(TOPK):
            m = jnp.max(mval)
            pick = jnp.min(jnp.where(mval == m, flat, BIG_I32))
            pick = jnp.where(m == NEG_INF, jnp.int32(0), pick)
            ok = m >= THR
            hit = jnp.logical_and(mval == m, flat == pick)
            mval = jnp.where(flat == pick, NEG_INF, mval)
            picks.append(pick)
            sims_o.append(jnp.where(ok, m, 0.0))
            vals_o.append(jnp.where(ok, 1.0, 0.0))
            selt = jnp.transpose(jnp.logical_and(hit, ok))
            krows.append(jnp.sum(jnp.where(selt, kcr, 0.0),
                                 axis=0, keepdims=True))
        picks_a.append(jnp.stack(picks).reshape(1, 1, TOPK))
        sims_a.append(jnp.stack(sims_o).reshape(1, 1, TOPK))
        vals_a.append(jnp.stack(vals_o).reshape(1, 1, TOPK))
        krows_a.append(jnp.concatenate(krows, axis=0).reshape(1, TOPK, D))
    tidx_ref[...] = jnp.concatenate(picks_a, axis=0)
    rsim_ref[...] = jnp.concatenate(sims_a, axis=0).astype(jnp.float32)
    valf_ref[...] = jnp.concatenate(vals_a, axis=0).astype(jnp.float32)
    rkeys_ref[...] = jnp.concatenate(krows_a, axis=0)


def _phase2(query, memory_keys, cids_flat):
    def mk_spec(j):
        return pl.BlockSpec((CHUNK, D), lambda i, ids, j=j: (ids[CCH * i + j], 0))

    grid_spec = pltpu.PrefetchScalarGridSpec(
        num_scalar_prefetch=1,
        grid=(NP2,),
        in_specs=[pl.BlockSpec((B, D), lambda i, ids: (0, 0))]
        + [mk_spec(j) for j in range(CCH)],
        out_specs=[
            pl.BlockSpec((RPS, 1, TOPK), lambda i, ids: (i, 0, 0)),
            pl.BlockSpec((RPS, 1, TOPK), lambda i, ids: (i, 0, 0)),
            pl.BlockSpec((RPS, 1, TOPK), lambda i, ids: (i, 0, 0)),
            pl.BlockSpec((RPS, TOPK, D), lambda i, ids: (i, 0, 0)),
        ],
        scratch_shapes=[pltpu.VMEM((B, CCH * CHUNK), jnp.float32)],
    )
    return pl.pallas_call(
        _p2_body,
        grid_spec=grid_spec,
        out_shape=[
            jax.ShapeDtypeStruct((B, 1, TOPK), jnp.int32),
            jax.ShapeDtypeStruct((B, 1, TOPK), jnp.float32),
            jax.ShapeDtypeStruct((B, 1, TOPK), jnp.float32),
            jax.ShapeDtypeStruct((B, TOPK, D), jnp.float32),
        ],
    )(cids_flat, query, *([memory_keys] * CCH))


RPV = 4
NPV = B // RPV
NRV = RPV * TOPK


def _p2v_body(ids_ref, *refs):
    v_refs = refs[:NRV]
    valf_ref, out_ref = refs[NRV:]
    rows = jnp.concatenate([r[0] for r in v_refs], axis=0)
    scale = jnp.transpose(valf_ref[...], (0, 2, 1))
    out_ref[...] = rows.reshape(RPV, TOPK, D) * scale


def _phase2v(memory_values, idx_flat, valf):
    vals3 = memory_values.reshape(CAP, 1, D)

    def mk_spec(j):
        return pl.BlockSpec((1, 1, D),
                            lambda i, ids, j=j: (ids[NRV * i + j], 0, 0))

    grid_spec = pltpu.PrefetchScalarGridSpec(
        num_scalar_prefetch=1,
        grid=(NPV,),
        in_specs=[mk_spec(j) for j in range(NRV)]
        + [pl.BlockSpec((RPV, 1, TOPK), lambda i, ids: (i, 0, 0))],
        out_specs=[pl.BlockSpec((RPV, TOPK, D), lambda i, ids: (i, 0, 0))],
    )
    return pl.pallas_call(
        _p2v_body,
        grid_spec=grid_spec,
        out_shape=[jax.ShapeDtypeStruct((B, TOPK, D), jnp.float32)],
    )(idx_flat, *([vals3] * NRV), valf)[0]


NSLOT = B * TOPK
SPW = NSLOT // 16
UROWS = CAP // 16
CROWS = UROWS // 16


def _extractf(vec, j, iota16):
    return jnp.sum(jnp.where(iota16 == j, vec, 0.0))


def _extracti(vec, j, iota16):
    return jnp.sum(jnp.where(iota16 == j, vec, 0))


def _p3_body(usage_hbm, idx_hbm, valid_hbm, uout_hbm,
             myidx_v, rowid_v, myval_v, urows_v, oh_v, mrows_v,
             cbuf_v, mask_sh, sem):
    core = lax.axis_index("c")
    sub = lax.axis_index("s")
    iota16 = lax.iota(jnp.int32, 16)

    @pl.when(core == 0)
    def _usage():
        pltpu.sync_copy(usage_hbm.at[pl.ds(sub * CROWS, CROWS)], cbuf_v)
        pltpu.sync_copy(cbuf_v, uout_hbm.at[pl.ds(sub * CROWS, CROWS)])

        pltpu.sync_copy(idx_hbm.at[pl.ds(sub * SPW, SPW)], myidx_v)
        pltpu.sync_copy(valid_hbm.at[pl.ds(sub * SPW, SPW)], myval_v)
        idx16 = myidx_v[...]
        vvec = myval_v[...]
        lanes = jnp.bitwise_and(idx16, 15)
        rowid_v[...] = lax.shift_right_logical(idx16, 4)
        pltpu.async_copy(usage_hbm.at[rowid_v], urows_v, sem).wait()

        zero16 = jnp.zeros((16,), jnp.float32)
        for j in range(SPW):
            oh_v[j, :] = zero16
        pltpu.sync_copy(oh_v, mask_sh.at[rowid_v])
        plsc.subcore_barrier()
        for j in range(SPW):
            lane = _extracti(lanes, j, iota16)
            vj = _extractf(vvec, j, iota16)
            oh_v[j, :] = jnp.where(iota16 == lane, vj, 0.0)
        pltpu.sync_copy(oh_v, mask_sh.at[rowid_v], add=True)
        plsc.subcore_barrier()
        pltpu.async_copy(mask_sh.at[rowid_v], mrows_v, sem).wait()

        for j in range(SPW):
            u = urows_v[j, :]
            upd = jnp.minimum(u * jnp.float32(1.1), jnp.float32(2.0))
            urows_v[j, :] = jnp.where(mrows_v[j, :] > 0.5, upd, u)
        pltpu.sync_copy(urows_v, uout_hbm.at[rowid_v])


def _phase3(usage2d, idx_flat, valid_flat):
    mesh = plsc.VectorSubcoreMesh(core_axis_name="c", subcore_axis_name="s")
    cp = pltpu.CompilerParams()
    for fld, val in (("needs_layout_passes", False),
                     ("use_tc_tiling_on_sc", False)):
        if fld in pltpu.CompilerParams.__dataclass_fields__:
            cp = dataclasses.replace(cp, **{fld: val})
    kfn = pl.kernel(
        _p3_body,
        mesh=mesh,
        compiler_params=cp,
        out_type=[
            jax.ShapeDtypeStruct((UROWS, 16), jnp.float32),
        ],
        scratch_types=[
            pltpu.VMEM((SPW,), jnp.int32),
            pltpu.VMEM((SPW,), jnp.int32),
            pltpu.VMEM((SPW,), jnp.float32),
            pltpu.VMEM((SPW, 16), jnp.float32),
            pltpu.VMEM((SPW, 16), jnp.float32),
            pltpu.VMEM((SPW, 16), jnp.float32),
            pltpu.VMEM((CROWS, 16), jnp.float32),
            pltpu.VMEM_SHARED((UROWS, 16), jnp.float32),
            pltpu.SemaphoreType.DMA,
        ],
    )
    return kfn(usage2d, idx_flat, valid_flat)[0]


def kernel(query, memory_keys, memory_values, memory_usage):
    cids8 = _phase1(query, memory_keys)
    cids_flat = jnp.transpose(cids8[:TOPK]).reshape(-1)
    tidx, rsim, valf, retrieved_keys = _phase2(query, memory_keys, cids_flat)
    retrieved_sims = rsim.reshape(B, TOPK)
    valid_flat = valf.reshape(-1)
    idx_flat = tidx.reshape(-1)
    retrieved_values = _phase2v(memory_values, idx_flat, valf)
    usage2d = memory_usage.reshape(UROWS, 16)
    uout = _phase3(usage2d, idx_flat, valid_flat)
    new_usage = uout.reshape(CAP)
    return retrieved_values, retrieved_keys, retrieved_sims, new_usage


# speedup vs baseline: 1.1902x; 1.0076x over previous
import dataclasses

import jax
import jax.numpy as jnp
from jax import lax
from jax.experimental import pallas as pl
from jax.experimental.pallas import tpu as pltpu
from jax.experimental.pallas import tpu_sc as plsc

CAP = 262144
D = 64
B = 64
TOPK = 4
THR = 0.1
EPS = 1e-8

KB = 8192
NSTEP = CAP // KB
CHUNK = 128
NCHUNK = CAP // CHUNK
CPS = KB // CHUNK
NCAND = TOPK * CHUNK

NEG_INF = float("-inf")
BIG_I32 = 1 << 30


def _normalize_rows(x):
    nrm = jnp.sqrt(jnp.sum(x * x, axis=1))
    return x / jnp.maximum(nrm, EPS)[:, None]


def _sims_bf16(kn, qn):
    return lax.dot_general(
        kn.astype(jnp.bfloat16), qn.astype(jnp.bfloat16),
        (((1,), (1,)), ((), ())), preferred_element_type=jnp.float32)


def _p1_body(q_ref, k_ref, cids_ref, acc_ref):
    i = pl.program_id(0)
    k = k_ref[...]
    kn = _normalize_rows(k)
    qn = _normalize_rows(q_ref[...])
    s = _sims_bf16(kn, qn)
    cm = jnp.max(s.reshape(CPS, CHUNK, B), axis=1)
    acc_ref[pl.ds(i * CPS, CPS), :] = cm

    @pl.when(i == NSTEP - 1)
    def _():
        a = acc_ref[...]
        rows = lax.broadcasted_iota(jnp.int32, (NCHUNK, B), 0)
        picks = []
        for _t in range(TOPK):
            m = jnp.max(a, axis=0)
            pick = jnp.min(jnp.where(a == m[None, :], rows, BIG_I32), axis=0)
            a = jnp.where(rows == pick[None, :], NEG_INF, a)
            picks.append(pick)
        picks += [jnp.zeros((B,), jnp.int32)] * 4
        cids_ref[...] = jnp.stack(picks, axis=0)


def _phase1(query, memory_keys):
    return pl.pallas_call(
        _p1_body,
        grid=(NSTEP,),
        in_specs=[
            pl.BlockSpec((B, D), lambda i: (0, 0)),
            pl.BlockSpec((KB, D), lambda i: (i, 0)),
        ],
        out_specs=pl.BlockSpec((8, B), lambda i: (0, 0)),
        out_shape=jax.ShapeDtypeStruct((8, B), jnp.int32),
        scratch_shapes=[pltpu.VMEM((NCHUNK, B), jnp.float32)],
    )(query, memory_keys)


RPS = 4
NP2 = B // RPS
CCH = RPS * TOPK


def _p2_body(ids_ref, q_ref, *refs):
    k_refs = refs[:CCH]
    tidx_ref, rsim_ref, valf_ref, rkeys_ref, s_ref = refs[CCH:]
    i = pl.program_id(0)
    kc = jnp.concatenate([r[...] for r in k_refs], axis=0)
    kn = _normalize_rows(kc)
    qn = _normalize_rows(q_ref[...])
    s_ref[...] = lax.dot_general(
        qn.astype(jnp.bfloat16), kn.astype(jnp.bfloat16),
        (((1,), (1,)), ((), ())), preferred_element_type=jnp.float32)

    lane = lax.broadcasted_iota(jnp.int32, (1, NCAND), 1)
    slot = lane >> 7
    offs = lane & (CHUNK - 1)
    picks_a, sims_a, vals_a, krows_a = [], [], [], []
    for r in range(RPS):
        val = s_ref[pl.ds(i * RPS + r, 1), pl.ds(r * NCAND, NCAND)]
        base = jnp.where(
            slot == 0, ids_ref[CCH * i + 4 * r],
            jnp.where(slot == 1, ids_ref[CCH * i + 4 * r + 1],
                      jnp.where(slot == 2, ids_ref[CCH * i + 4 * r + 2],
                                ids_ref[CCH * i + 4 * r + 3])))
        flat = base * CHUNK + offs
        kcr = kc[r * NCAND:(r + 1) * NCAND]

        mval = jnp.where(val >= THR, val, NEG_INF)
        picks, sims_o, vals_o, krows = [], [], [], []
        for _t in range(TOPK):
            m = jnp.max(mval)
            pick = jnp.min(jnp.where(mval == m, flat, BIG_I32))
            pick = jnp.where(m == NEG_INF, jnp.int32(0), pick)
            ok = m >= THR
            hit = jnp.logical_and(mval == m, flat == pick)
            mval = jnp.where(flat == pick, NEG_INF, mval)
            picks.append(pick)
            sims_o.append(jnp.where(ok, m, 0.0))
            vals_o.append(jnp.where(ok, 1.0, 0.0))
            selt = jnp.transpose(jnp.logical_and(hit, ok))
            krows.append(jnp.sum(jnp.where(selt, kcr, 0.0),
                                 axis=0, keepdims=True))
        picks_a.append(jnp.stack(picks).reshape(1, 1, TOPK))
        sims_a.append(jnp.stack(sims_o).reshape(1, 1, TOPK))
        vals_a.append(jnp.stack(vals_o).reshape(1, 1, TOPK))
        krows_a.append(jnp.concatenate(krows, axis=0).reshape(1, TOPK, D))
    tidx_ref[...] = jnp.concatenate(picks_a, axis=0)
    rsim_ref[...] = jnp.concatenate(sims_a, axis=0).astype(jnp.float32)
    valf_ref[...] = jnp.concatenate(vals_a, axis=0).astype(jnp.float32)
    rkeys_ref[...] = jnp.concatenate(krows_a, axis=0)


def _phase2(query, memory_keys, cids_flat):
    def mk_spec(j):
        return pl.BlockSpec((CHUNK, D), lambda i, ids, j=j: (ids[CCH * i + j], 0))

    grid_spec = pltpu.PrefetchScalarGridSpec(
        num_scalar_prefetch=1,
        grid=(NP2,),
        in_specs=[pl.BlockSpec((B, D), lambda i, ids: (0, 0))]
        + [mk_spec(j) for j in range(CCH)],
        out_specs=[
            pl.BlockSpec((RPS, 1, TOPK), lambda i, ids: (i, 0, 0)),
            pl.BlockSpec((RPS, 1, TOPK), lambda i, ids: (i, 0, 0)),
            pl.BlockSpec((RPS, 1, TOPK), lambda i, ids: (i, 0, 0)),
            pl.BlockSpec((RPS, TOPK, D), lambda i, ids: (i, 0, 0)),
        ],
        scratch_shapes=[pltpu.VMEM((B, CCH * CHUNK), jnp.float32)],
    )
    return pl.pallas_call(
        _p2_body,
        grid_spec=grid_spec,
        out_shape=[
            jax.ShapeDtypeStruct((B, 1, TOPK), jnp.int32),
            jax.ShapeDtypeStruct((B, 1, TOPK), jnp.float32),
            jax.ShapeDtypeStruct((B, 1, TOPK), jnp.float32),
            jax.ShapeDtypeStruct((B, TOPK, D), jnp.float32),
        ],
    )(cids_flat, query, *([memory_keys] * CCH))


RPV = 8
NPV = B // RPV
NRV = RPV * TOPK


def _p2v_body(ids_ref, *refs):
    v_refs = refs[:NRV]
    valf_ref, out_ref = refs[NRV:]
    rows = jnp.concatenate([r[0] for r in v_refs], axis=0)
    scale = jnp.transpose(valf_ref[...], (0, 2, 1))
    out_ref[...] = rows.reshape(RPV, TOPK, D) * scale


def _phase2v(memory_values, idx_flat, valf):
    vals3 = memory_values.reshape(CAP, 1, D)

    def mk_spec(j):
        return pl.BlockSpec((1, 1, D),
                            lambda i, ids, j=j: (ids[NRV * i + j], 0, 0))

    grid_spec = pltpu.PrefetchScalarGridSpec(
        num_scalar_prefetch=1,
        grid=(NPV,),
        in_specs=[mk_spec(j) for j in range(NRV)]
        + [pl.BlockSpec((RPV, 1, TOPK), lambda i, ids: (i, 0, 0))],
        out_specs=[pl.BlockSpec((RPV, TOPK, D), lambda i, ids: (i, 0, 0))],
    )
    return pl.pallas_call(
        _p2v_body,
        grid_spec=grid_spec,
        out_shape=[jax.ShapeDtypeStruct((B, TOPK, D), jnp.float32)],
    )(idx_flat, *([vals3] * NRV), valf)[0]


NSLOT = B * TOPK
SPW = NSLOT // 16
UROWS = CAP // 16
CROWS = UROWS // 16


def _extractf(vec, j, iota16):
    return jnp.sum(jnp.where(iota16 == j, vec, 0.0))


def _extracti(vec, j, iota16):
    return jnp.sum(jnp.where(iota16 == j, vec, 0))


def _p3_body(usage_hbm, idx_hbm, valid_hbm, uout_hbm,
             myidx_v, rowid_v, myval_v, urows_v, oh_v, mrows_v,
             cbuf_v, mask_sh, sem):
    core = lax.axis_index("c")
    sub = lax.axis_index("s")
    iota16 = lax.iota(jnp.int32, 16)

    @pl.when(core == 0)
    def _usage():
        pltpu.sync_copy(usage_hbm.at[pl.ds(sub * CROWS, CROWS)], cbuf_v)
        pltpu.sync_copy(cbuf_v, uout_hbm.at[pl.ds(sub * CROWS, CROWS)])

        pltpu.sync_copy(idx_hbm.at[pl.ds(sub * SPW, SPW)], myidx_v)
        pltpu.sync_copy(valid_hbm.at[pl.ds(sub * SPW, SPW)], myval_v)
        idx16 = myidx_v[...]
        vvec = myval_v[...]
        lanes = jnp.bitwise_and(idx16, 15)
        rowid_v[...] = lax.shift_right_logical(idx16, 4)
        pltpu.async_copy(usage_hbm.at[rowid_v], urows_v, sem).wait()

        zero16 = jnp.zeros((16,), jnp.float32)
        for j in range(SPW):
            oh_v[j, :] = zero16
        pltpu.sync_copy(oh_v, mask_sh.at[rowid_v])
        plsc.subcore_barrier()
        for j in range(SPW):
            lane = _extracti(lanes, j, iota16)
            vj = _extractf(vvec, j, iota16)
            oh_v[j, :] = jnp.where(iota16 == lane, vj, 0.0)
        pltpu.sync_copy(oh_v, mask_sh.at[rowid_v], add=True)
        plsc.subcore_barrier()
        pltpu.async_copy(mask_sh.at[rowid_v], mrows_v, sem).wait()

        for j in range(SPW):
            u = urows_v[j, :]
            upd = jnp.minimum(u * jnp.float32(1.1), jnp.float32(2.0))
            urows_v[j, :] = jnp.where(mrows_v[j, :] > 0.5, upd, u)
        pltpu.sync_copy(urows_v, uout_hbm.at[rowid_v])


def _phase3(usage2d, idx_flat, valid_flat):
    mesh = plsc.VectorSubcoreMesh(core_axis_name="c", subcore_axis_name="s")
    cp = pltpu.CompilerParams()
    for fld, val in (("needs_layout_passes", False),
                     ("use_tc_tiling_on_sc", False)):
        if fld in pltpu.CompilerParams.__dataclass_fields__:
            cp = dataclasses.replace(cp, **{fld: val})
    kfn = pl.kernel(
        _p3_body,
        mesh=mesh,
        compiler_params=cp,
        out_type=[
            jax.ShapeDtypeStruct((UROWS, 16), jnp.float32),
        ],
        scratch_types=[
            pltpu.VMEM((SPW,), jnp.int32),
            pltpu.VMEM((SPW,), jnp.int32),
            pltpu.VMEM((SPW,), jnp.float32),
            pltpu.VMEM((SPW, 16), jnp.float32),
            pltpu.VMEM((SPW, 16), jnp.float32),
            pltpu.VMEM((SPW, 16), jnp.float32),
            pltpu.VMEM((CROWS, 16), jnp.float32),
            pltpu.VMEM_SHARED((UROWS, 16), jnp.float32),
            pltpu.SemaphoreType.DMA,
        ],
    )
    return kfn(usage2d, idx_flat, valid_flat)[0]


def kernel(query, memory_keys, memory_values, memory_usage):
    cids8 = _phase1(query, memory_keys)
    cids_flat = jnp.transpose(cids8[:TOPK]).reshape(-1)
    tidx, rsim, valf, retrieved_keys = _phase2(query, memory_keys, cids_flat)
    retrieved_sims = rsim.reshape(B, TOPK)
    valid_flat = valf.reshape(-1)
    idx_flat = tidx.reshape(-1)
    retrieved_values = _phase2v(memory_values, idx_flat, valf)
    usage2d = memory_usage.reshape(UROWS, 16)
    uout = _phase3(usage2d, idx_flat, valid_flat)
    new_usage = uout.reshape(CAP)
    return retrieved_values, retrieved_keys, retrieved_sims, new_usage
